# baseline (speedup 1.0000x reference)
import functools

import jax
import jax.numpy as jnp
from jax import lax
from jax.experimental import pallas as pl
from jax.experimental.pallas import tpu as pltpu
from jax.experimental.pallas import tpu_sc as plsc

_GRID_PER_BATCH = 1024
_R = 512
_MPAD = 10240
_C = 640
_NCH = _MPAD // _C
_MASKVAL = 1e10


def _top3_body(sref, gp_ref, mp_ref, xn_ref, io_ref, idx_ref, w_ref):
    b = pl.program_id(0) // (_GRID_PER_BATCH // _R)
    cs = sref[2 * b]
    ncs = sref[2 * b + 1]
    gp = gp_ref[...]
    gpb = gp.astype(jnp.bfloat16)
    yn = ((gp[:, 0:1] * gp[:, 0:1] + gp[:, 2:3] * gp[:, 2:3])
          + gp[:, 1:2] * gp[:, 1:2])

    def chunk_body(cc, carry):
        rv0, rv1, rv2, rx0, rx1, rx2 = carry
        mp = mp_ref[pl.ds(cc * _C, _C), :]
        mpb = mp[:, 0:3].astype(jnp.bfloat16)
        p = lax.dot_general(gpb, mpb, (((1,), (1,)), ((), ())),
                            preferred_element_type=jnp.float32)
        xn = xn_ref[pl.ds(cc * 8, 8), :][0:1, :]
        d2 = (yn + xn) - 2.0 * p
        iota = io_ref[pl.ds(cc * 8, 8), :][0:1, :]
        nv, nx = [], []
        for j in range(3):
            m = jnp.min(d2, axis=1, keepdims=True)
            sel = jnp.where(d2 == m, iota, jnp.float32(3e38))
            ij = jnp.min(sel, axis=1, keepdims=True)
            nv.append(m)
            nx.append(ij)
            if j < 2:
                d2 = jnp.where(iota == ij, jnp.float32(3e38), d2)
        c0 = nv[0] < rv0
        o0v = jnp.where(c0, nv[0], rv0)
        o0x = jnp.where(c0, nx[0], rx0)
        c1 = nv[1] < rv0
        c2 = nv[0] < rv1
        o1v = jnp.where(c0, jnp.where(c1, nv[1], rv0),
                        jnp.where(c2, nv[0], rv1))
        o1x = jnp.where(c0, jnp.where(c1, nx[1], rx0),
                        jnp.where(c2, nx[0], rx1))
        c3 = nv[2] < rv0
        c4 = nv[1] < rv1
        c5 = nv[0] < rv2
        i2v = jnp.where(c4, nv[1], rv1)
        i2x = jnp.where(c4, nx[1], rx1)
        o2v = jnp.where(c0,
                        jnp.where(c1, jnp.where(c3, nv[2], rv0), i2v),
                        jnp.where(c2, i2v, jnp.where(c5, nv[0], rv2)))
        o2x = jnp.where(c0,
                        jnp.where(c1, jnp.where(c3, nx[2], rx0), i2x),
                        jnp.where(c2, i2x, jnp.where(c5, nx[0], rx2)))
        return (o0v, o1v, o2v, o0x, o1x, o2x)

    big = jnp.full((_R, 1), 3e38, jnp.float32)
    zero = jnp.zeros((_R, 1), jnp.float32)
    v0, v1, v2, x0, x1, x2 = lax.fori_loop(
        cs, cs + ncs, chunk_body, (big, big, big, zero, zero, zero))

    ws = [1.0 / jnp.maximum(vv, jnp.float32(1e-16)) for vv in (v0, v1, v2)]
    den = ws[0] + ws[1] + ws[2]
    wn = [w / den for w in ws]
    li3 = lax.broadcasted_iota(jnp.int32, (_R, 3), 1)
    idx_ref[...] = jnp.where(li3 == 0, x0,
                             jnp.where(li3 == 1, x1, x2)).astype(jnp.int32)
    lg = lax.broadcasted_iota(jnp.int32, (_R, 48), 1) // 16
    w_ref[...] = jnp.where(lg == 0, wn[0],
                           jnp.where(lg == 1, wn[1], wn[2]))


def _run_top3(scal, grid_pos, mp8, xnb, iob):
    n_grid = grid_pos.shape[0]
    bpb = _GRID_PER_BATCH // _R
    grid_spec = pltpu.PrefetchScalarGridSpec(
        num_scalar_prefetch=1,
        grid=(n_grid // _R,),
        in_specs=[
            pl.BlockSpec((_R, 3), lambda i, s: (i, 0)),
            pl.BlockSpec((_MPAD, 8), lambda i, s: (0, 0)),
            pl.BlockSpec((8 * _NCH, _C), lambda i, s: (i // bpb, 0)),
            pl.BlockSpec((8 * _NCH, _C), lambda i, s: (0, 0)),
        ],
        out_specs=[
            pl.BlockSpec((_R, 3), lambda i, s: (i, 0)),
            pl.BlockSpec((_R, 48), lambda i, s: (i, 0)),
        ],
    )
    return pl.pallas_call(
        _top3_body,
        grid_spec=grid_spec,
        out_shape=[
            jax.ShapeDtypeStruct((n_grid, 3), jnp.int32),
            jax.ShapeDtypeStruct((n_grid, 48), jnp.float32),
        ],
    )(scal, grid_pos, mp8, xnb, iob)


_NW = 32
_PTS_PER_W = 128
_CHUNK = 32
_NCHK = _PTS_PER_W // _CHUNK


def _sc_gather_body(x_hbm, idx_hbm, w_hbm, out_hbm, idx_v, w_v, rows0, rows1,
                    out_v, sem0, sem1):
    c = lax.axis_index("c")
    s = lax.axis_index("s")
    wid = s * 2 + c
    base = wid * _PTS_PER_W
    pltpu.sync_copy(idx_hbm.at[pl.ds(base * 3, 3 * _PTS_PER_W)], idx_v)
    pltpu.sync_copy(w_hbm.at[pl.ds(base * 48, 48 * _PTS_PER_W)], w_v)
    bufs = [(rows0, sem0), (rows1, sem1)]
    copies = [None] * _NCHK
    copies[0] = pltpu.async_copy(
        x_hbm.at[idx_v.at[pl.ds(0, 3 * _CHUNK)]], rows0, sem0)
    for ch in range(_NCHK):
        if ch + 1 < _NCHK:
            nbuf, nsem = bufs[(ch + 1) % 2]
            copies[ch + 1] = pltpu.async_copy(
                x_hbm.at[idx_v.at[pl.ds((ch + 1) * 3 * _CHUNK, 3 * _CHUNK)]],
                nbuf, nsem)
        cur, _ = bufs[ch % 2]
        copies[ch].wait()

        def body(p, carry, cur=cur, ch=ch):
            rb = p * 3
            o = ch * _CHUNK + p
            w0 = w_v[pl.ds(o * 48, 16)]
            w1 = w_v[pl.ds(o * 48 + 16, 16)]
            w2 = w_v[pl.ds(o * 48 + 32, 16)]
            for v in range(16):
                col = pl.ds(v * 16, 16)
                out_v[o, col] = (w0 * cur[rb, col]
                                 + w1 * cur[rb + 1, col]
                                 + w2 * cur[rb + 2, col])
            return carry

        lax.fori_loop(0, _CHUNK, body, 0)
    pltpu.sync_copy(out_v, out_hbm.at[pl.ds(base, _PTS_PER_W)])


def _run_sc_gather(x, flat_idx, flat_w, n_grid, d_feat):
    mesh = plsc.VectorSubcoreMesh(core_axis_name="c", subcore_axis_name="s")
    k = functools.partial(
        pl.kernel,
        mesh=mesh,
        out_type=jax.ShapeDtypeStruct((n_grid, d_feat), jnp.float32),
        scratch_types=[
            pltpu.VMEM((3 * _PTS_PER_W,), jnp.int32),
            pltpu.VMEM((48 * _PTS_PER_W,), jnp.float32),
            pltpu.VMEM((3 * _CHUNK, d_feat), jnp.float32),
            pltpu.VMEM((3 * _CHUNK, d_feat), jnp.float32),
            pltpu.VMEM((_PTS_PER_W, d_feat), jnp.float32),
            pltpu.SemaphoreType.DMA,
            pltpu.SemaphoreType.DMA,
        ],
    )(_sc_gather_body)
    return k(x, flat_idx, flat_w)


def kernel(x, mesh_pos, grid_pos, batch_idx):
    n_mesh, d_feat = x.shape
    n_grid = grid_pos.shape[0]
    n_batch = n_grid // _GRID_PER_BATCH
    bidx = batch_idx.astype(jnp.int32)

    mp8 = jnp.zeros((_MPAD, 8), jnp.float32).at[:n_mesh, :3].set(mesh_pos)
    xn = ((mesh_pos[:, 0] * mesh_pos[:, 0] + mesh_pos[:, 2] * mesh_pos[:, 2])
          + mesh_pos[:, 1] * mesh_pos[:, 1])
    xnp = jnp.full((_MPAD,), jnp.float32(_MASKVAL)).at[:n_mesh].set(xn)
    bip = jnp.full((_MPAD,), -1, jnp.int32).at[:n_mesh].set(bidx)
    batches = jnp.arange(n_batch, dtype=jnp.int32)
    xnb = jnp.where(bip[None, :] == batches[:, None], xnp[None, :],
                    jnp.float32(_MASKVAL))
    xnb = jnp.repeat(xnb.reshape(n_batch, _NCH, _C), 8,
                     axis=1).reshape(n_batch * 8 * _NCH, _C)
    iob = jnp.repeat(
        jnp.arange(_MPAD, dtype=jnp.float32).reshape(_NCH, _C), 8,
        axis=0)

    cmp = bidx[:, None]
    start = jnp.sum(cmp < batches[None, :], axis=0, dtype=jnp.int32)
    end = jnp.sum(cmp <= batches[None, :], axis=0, dtype=jnp.int32)
    nb = end - start
    cs = jnp.where(nb < 3, 0, start // _C)
    ce = jnp.where(nb == 0, 0, jnp.maximum(end - 1, start) // _C)
    ncs = ce - cs + 1
    scal = jnp.stack([cs, ncs], axis=1).reshape(-1)

    idx, wb = _run_top3(scal, grid_pos, mp8, xnb, iob)
    flat_idx = idx.reshape(-1)
    flat_w = wb.reshape(-1)
    return _run_sc_gather(x, flat_idx, flat_w, n_grid, d_feat)


# speedup vs baseline: 1.0529x; 1.0529x over previous
import functools

import jax
import jax.numpy as jnp
from jax import lax
from jax.experimental import pallas as pl
from jax.experimental.pallas import tpu as pltpu
from jax.experimental.pallas import tpu_sc as plsc

_GRID_PER_BATCH = 1024
_R = 1024
_MPAD = 10240
_C = 1280
_NCH = _MPAD // _C
_MASKVAL = 1e10


def _top3_body(sref, gp_ref, mp_ref, xn_ref, io_ref, idx_ref, w_ref):
    b = pl.program_id(0) // (_GRID_PER_BATCH // _R)
    cs = sref[2 * b]
    ncs = sref[2 * b + 1]
    gp = gp_ref[...]
    gpb = gp.astype(jnp.bfloat16)
    yn = ((gp[:, 0:1] * gp[:, 0:1] + gp[:, 2:3] * gp[:, 2:3])
          + gp[:, 1:2] * gp[:, 1:2])

    def chunk_body(cc, carry):
        rv0, rv1, rv2, rx0, rx1, rx2 = carry
        mp = mp_ref[pl.ds(cc * _C, _C), :]
        mpb = mp[:, 0:3].astype(jnp.bfloat16)
        p = lax.dot_general(gpb, mpb, (((1,), (1,)), ((), ())),
                            preferred_element_type=jnp.float32)
        xn = xn_ref[pl.ds(cc * 8, 8), :][0:1, :]
        d2 = (yn + xn) - 2.0 * p
        iota = io_ref[pl.ds(cc * 8, 8), :][0:1, :]
        nv, nx = [], []
        for j in range(3):
            m = jnp.min(d2, axis=1, keepdims=True)
            sel = jnp.where(d2 == m, iota, jnp.float32(3e38))
            ij = jnp.min(sel, axis=1, keepdims=True)
            nv.append(m)
            nx.append(ij)
            if j < 2:
                d2 = jnp.where(iota == ij, jnp.float32(3e38), d2)
        c0 = nv[0] < rv0
        o0v = jnp.where(c0, nv[0], rv0)
        o0x = jnp.where(c0, nx[0], rx0)
        c1 = nv[1] < rv0
        c2 = nv[0] < rv1
        o1v = jnp.where(c0, jnp.where(c1, nv[1], rv0),
                        jnp.where(c2, nv[0], rv1))
        o1x = jnp.where(c0, jnp.where(c1, nx[1], rx0),
                        jnp.where(c2, nx[0], rx1))
        c3 = nv[2] < rv0
        c4 = nv[1] < rv1
        c5 = nv[0] < rv2
        i2v = jnp.where(c4, nv[1], rv1)
        i2x = jnp.where(c4, nx[1], rx1)
        o2v = jnp.where(c0,
                        jnp.where(c1, jnp.where(c3, nv[2], rv0), i2v),
                        jnp.where(c2, i2v, jnp.where(c5, nv[0], rv2)))
        o2x = jnp.where(c0,
                        jnp.where(c1, jnp.where(c3, nx[2], rx0), i2x),
                        jnp.where(c2, i2x, jnp.where(c5, nx[0], rx2)))
        return (o0v, o1v, o2v, o0x, o1x, o2x)

    big = jnp.full((_R, 1), 3e38, jnp.float32)
    zero = jnp.zeros((_R, 1), jnp.float32)
    v0, v1, v2, x0, x1, x2 = lax.fori_loop(
        cs, cs + ncs, chunk_body, (big, big, big, zero, zero, zero))

    ws = [1.0 / jnp.maximum(vv, jnp.float32(1e-16)) for vv in (v0, v1, v2)]
    den = ws[0] + ws[1] + ws[2]
    wn = [w / den for w in ws]
    li3 = lax.broadcasted_iota(jnp.int32, (_R, 3), 1)
    idx_ref[...] = jnp.where(li3 == 0, x0,
                             jnp.where(li3 == 1, x1, x2)).astype(jnp.int32)
    lg = lax.broadcasted_iota(jnp.int32, (_R, 48), 1) // 16
    w_ref[...] = jnp.where(lg == 0, wn[0],
                           jnp.where(lg == 1, wn[1], wn[2]))


def _run_top3(scal, grid_pos, mp8, xnb, iob):
    n_grid = grid_pos.shape[0]
    bpb = _GRID_PER_BATCH // _R
    grid_spec = pltpu.PrefetchScalarGridSpec(
        num_scalar_prefetch=1,
        grid=(n_grid // _R,),
        in_specs=[
            pl.BlockSpec((_R, 3), lambda i, s: (i, 0)),
            pl.BlockSpec((_MPAD, 8), lambda i, s: (0, 0)),
            pl.BlockSpec((8 * _NCH, _C), lambda i, s: (i // bpb, 0)),
            pl.BlockSpec((8 * _NCH, _C), lambda i, s: (0, 0)),
        ],
        out_specs=[
            pl.BlockSpec((_R, 3), lambda i, s: (i, 0)),
            pl.BlockSpec((_R, 48), lambda i, s: (i, 0)),
        ],
    )
    return pl.pallas_call(
        _top3_body,
        grid_spec=grid_spec,
        out_shape=[
            jax.ShapeDtypeStruct((n_grid, 3), jnp.int32),
            jax.ShapeDtypeStruct((n_grid, 48), jnp.float32),
        ],
    )(scal, grid_pos, mp8, xnb, iob)


_NW = 32
_PTS_PER_W = 128
_CHUNK = 32
_NCHK = _PTS_PER_W // _CHUNK


def _sc_gather_body(x_hbm, idx_hbm, w_hbm, out_hbm, idx_v, w_v, rows0, rows1,
                    out_v, sem0, sem1):
    c = lax.axis_index("c")
    s = lax.axis_index("s")
    wid = s * 2 + c
    base = wid * _PTS_PER_W
    pltpu.sync_copy(idx_hbm.at[pl.ds(base * 3, 3 * _PTS_PER_W)], idx_v)
    pltpu.sync_copy(w_hbm.at[pl.ds(base * 48, 48 * _PTS_PER_W)], w_v)
    bufs = [(rows0, sem0), (rows1, sem1)]
    copies = [None] * _NCHK
    copies[0] = pltpu.async_copy(
        x_hbm.at[idx_v.at[pl.ds(0, 3 * _CHUNK)]], rows0, sem0)
    for ch in range(_NCHK):
        if ch + 1 < _NCHK:
            nbuf, nsem = bufs[(ch + 1) % 2]
            copies[ch + 1] = pltpu.async_copy(
                x_hbm.at[idx_v.at[pl.ds((ch + 1) * 3 * _CHUNK, 3 * _CHUNK)]],
                nbuf, nsem)
        cur, _ = bufs[ch % 2]
        copies[ch].wait()

        def body(p, carry, cur=cur, ch=ch):
            rb = p * 3
            o = ch * _CHUNK + p
            w0 = w_v[pl.ds(o * 48, 16)]
            w1 = w_v[pl.ds(o * 48 + 16, 16)]
            w2 = w_v[pl.ds(o * 48 + 32, 16)]
            for v in range(16):
                col = pl.ds(v * 16, 16)
                out_v[o, col] = (w0 * cur[rb, col]
                                 + w1 * cur[rb + 1, col]
                                 + w2 * cur[rb + 2, col])
            return carry

        lax.fori_loop(0, _CHUNK, body, 0)
    pltpu.sync_copy(out_v, out_hbm.at[pl.ds(base, _PTS_PER_W)])


def _run_sc_gather(x, flat_idx, flat_w, n_grid, d_feat):
    mesh = plsc.VectorSubcoreMesh(core_axis_name="c", subcore_axis_name="s")
    k = functools.partial(
        pl.kernel,
        mesh=mesh,
        out_type=jax.ShapeDtypeStruct((n_grid, d_feat), jnp.float32),
        scratch_types=[
            pltpu.VMEM((3 * _PTS_PER_W,), jnp.int32),
            pltpu.VMEM((48 * _PTS_PER_W,), jnp.float32),
            pltpu.VMEM((3 * _CHUNK, d_feat), jnp.float32),
            pltpu.VMEM((3 * _CHUNK, d_feat), jnp.float32),
            pltpu.VMEM((_PTS_PER_W, d_feat), jnp.float32),
            pltpu.SemaphoreType.DMA,
            pltpu.SemaphoreType.DMA,
        ],
    )(_sc_gather_body)
    return k(x, flat_idx, flat_w)


def kernel(x, mesh_pos, grid_pos, batch_idx):
    n_mesh, d_feat = x.shape
    n_grid = grid_pos.shape[0]
    n_batch = n_grid // _GRID_PER_BATCH
    bidx = batch_idx.astype(jnp.int32)

    mp8 = jnp.zeros((_MPAD, 8), jnp.float32).at[:n_mesh, :3].set(mesh_pos)
    xn = ((mesh_pos[:, 0] * mesh_pos[:, 0] + mesh_pos[:, 2] * mesh_pos[:, 2])
          + mesh_pos[:, 1] * mesh_pos[:, 1])
    xnp = jnp.full((_MPAD,), jnp.float32(_MASKVAL)).at[:n_mesh].set(xn)
    bip = jnp.full((_MPAD,), -1, jnp.int32).at[:n_mesh].set(bidx)
    batches = jnp.arange(n_batch, dtype=jnp.int32)
    xnb = jnp.where(bip[None, :] == batches[:, None], xnp[None, :],
                    jnp.float32(_MASKVAL))
    xnb = jnp.repeat(xnb.reshape(n_batch, _NCH, _C), 8,
                     axis=1).reshape(n_batch * 8 * _NCH, _C)
    iob = jnp.repeat(
        jnp.arange(_MPAD, dtype=jnp.float32).reshape(_NCH, _C), 8,
        axis=0)

    cmp = bidx[:, None]
    start = jnp.sum(cmp < batches[None, :], axis=0, dtype=jnp.int32)
    end = jnp.sum(cmp <= batches[None, :], axis=0, dtype=jnp.int32)
    nb = end - start
    cs = jnp.where(nb < 3, 0, start // _C)
    ce = jnp.where(nb == 0, 0, jnp.maximum(end - 1, start) // _C)
    ncs = ce - cs + 1
    scal = jnp.stack([cs, ncs], axis=1).reshape(-1)

    idx, wb = _run_top3(scal, grid_pos, mp8, xnb, iob)
    flat_idx = idx.reshape(-1)
    flat_w = wb.reshape(-1)
    return _run_sc_gather(x, flat_idx, flat_w, n_grid, d_feat)
